# Initial kernel scaffold; baseline (speedup 1.0000x reference)
import jax
import jax.numpy as jnp
from jax.experimental import pallas as pl


def kernel(z, t, conditioning, mask, params):
    raise NotImplementedError("write your pallas kernel here")


# speedup vs baseline: 10.8102x; 10.8102x over previous
import functools

import jax
import jax.numpy as jnp
import numpy as np
from jax import lax
from jax.experimental import pallas as pl
from jax.experimental.pallas import tpu as pltpu
from jax.experimental.pallas import tpu_sc as plsc

B, N, D = 4, 1024, 3
K = 20
LAT = 128
D_T = 32
D_COND = 40
E = N * K
BN = B * N
BE = B * E
NW = 32
PER_W = BE // NW
CH = 128
_INTERP = False


def _fs(shape):
    return pl.BlockSpec(shape, lambda *_: (0,) * len(shape))


def _k0_body(t_ref, cond_ref, freqs_ref, w0, b0, w1, b1, w2, b2,
             wg, beg, ag, bng, ecg_ref, ncg_ref):
    t = t_ref[...]
    args = t * freqs_ref[...]
    x = jnp.concatenate([jnp.sin(args), jnp.cos(args), cond_ref[...]], axis=1)
    x = jax.nn.gelu(jnp.dot(x, w0[...], preferred_element_type=jnp.float32) + b0[...])
    x = jax.nn.gelu(jnp.dot(x, w1[...], preferred_element_type=jnp.float32) + b1[...])
    g = jnp.dot(x, w2[...], preferred_element_type=jnp.float32) + b2[...]
    for s in range(3):
        ecg_ref[s] = jnp.dot(g, wg[s], preferred_element_type=jnp.float32) + beg[s]
        ncg_ref[s] = jnp.dot(g, ag[s], preferred_element_type=jnp.float32) + bng[s]


def _k0(t2, cond, freqs, cw, wg, beg, ag, bng):
    (w0, b0), (w1, b1), (w2, b2) = cw
    out_shape = (jax.ShapeDtypeStruct((3, B, LAT), jnp.float32),
                 jax.ShapeDtypeStruct((3, B, LAT), jnp.float32))
    args = (t2, cond, freqs, w0, b0, w1, b1, w2, b2, wg, beg, ag, bng)
    return pl.pallas_call(
        _k0_body,
        out_shape=out_shape,
        in_specs=[_fs(a.shape) for a in args],
        out_specs=(_fs((3, B, LAT)), _fs((3, B, LAT))),
        interpret=_INTERP,
    )(*args)


def _k1a_body(z_ref, m_ref, n0w, n0b, n1w, n1b, ws0, wr0,
              idx_ref, de_ref, h_ref, hs_ref, hr_ref):
    g = pl.program_id(0)
    x = z_ref[0]
    sqc = jnp.sum(x * x, axis=1, keepdims=True)
    sqr = jnp.transpose(sqc)
    xx = lax.dot_general(x, x, (((1,), (1,)), ((), ())),
                         preferred_element_type=jnp.float32)
    dist = sqc + sqr - 2.0 * xx
    m_col = jnp.transpose(m_ref[0])
    dist = jnp.where(m_col > 0, dist, 1e10)
    s1 = jnp.dot(x, jnp.ones((3, 1), jnp.float32),
                 preferred_element_type=jnp.float32)
    s1_row = jnp.transpose(s1)
    sub = lax.broadcasted_iota(jnp.int32, (N, N), 0)
    big_i = jnp.int32(2 ** 30)
    for k in range(K):
        mn = jnp.min(dist, axis=0, keepdims=True)
        amin = jnp.min(jnp.where(dist == mn, sub, big_i), axis=0,
                       keepdims=True)
        sel = sub == amin
        tgt_s = jnp.sum(jnp.where(sel, s1, 0.0), axis=0, keepdims=True)
        idx_ref[0, k] = (amin + g * N)[0]
        de_ref[0, k] = (s1_row - tgt_s)[0]
        dist = jnp.where(sel, 1e10, dist)
    h = jax.nn.gelu(jnp.dot(x, n0w[...], preferred_element_type=jnp.float32) + n0b[...])
    h = jnp.dot(h, n1w[...], preferred_element_type=jnp.float32) + n1b[...]
    h_ref[0] = h
    hs_ref[0] = jnp.dot(h, ws0[...], preferred_element_type=jnp.float32)
    hr_ref[0] = jnp.dot(h, wr0[...], preferred_element_type=jnp.float32)


def _k1a(z, mask3, new, ws0, wr0):
    (n0w, n0b), (n1w, n1b) = new
    out_shape = (jax.ShapeDtypeStruct((B, K, N), jnp.int32),
                 jax.ShapeDtypeStruct((B, K, N), jnp.float32),
                 jax.ShapeDtypeStruct((B, N, LAT), jnp.float32),
                 jax.ShapeDtypeStruct((B, N, LAT), jnp.float32),
                 jax.ShapeDtypeStruct((B, N, LAT), jnp.float32))
    wspecs = [_fs(w.shape) for w in (n0w, n0b, n1w, n1b, ws0, wr0)]
    return pl.pallas_call(
        _k1a_body,
        grid=(B,),
        out_shape=out_shape,
        in_specs=[pl.BlockSpec((1, N, D), lambda g: (g, 0, 0)),
                  pl.BlockSpec((1, 1, N), lambda g: (g, 0, 0))] + wspecs,
        out_specs=(pl.BlockSpec((1, K, N), lambda g: (g, 0, 0)),
                   pl.BlockSpec((1, K, N), lambda g: (g, 0, 0)),
                   pl.BlockSpec((1, N, LAT), lambda g: (g, 0, 0)),
                   pl.BlockSpec((1, N, LAT), lambda g: (g, 0, 0)),
                   pl.BlockSpec((1, N, LAT), lambda g: (g, 0, 0))),
        interpret=_INTERP,
    )(z, mask3, n0w, n0b, n1w, n1b, ws0, wr0)


def _k1b_body(d_ref, w1row, b1row, w2, b2, e_ref):
    cols = []
    for r in range(8):
        cols.append(jnp.transpose(d_ref[0, r:r + 1, :]))
    v = jnp.concatenate(cols, axis=0)
    a = jax.nn.gelu(v * w1row[...] + b1row[...])
    e_ref[0] = jnp.dot(a, w2[...], preferred_element_type=jnp.float32) + b2[...]


def _k1b(d3, w1row, b1row, w2, b2):
    return pl.pallas_call(
        _k1b_body,
        grid=(B, K),
        out_shape=jax.ShapeDtypeStruct((B, E, LAT), jnp.float32),
        in_specs=[pl.BlockSpec((1, 8, CH), lambda g, k: (g, k, 0)),
                  _fs(w1row.shape), _fs(b1row.shape), _fs(w2.shape), _fs(b2.shape)],
        out_specs=pl.BlockSpec((1, N, LAT), lambda g, k: (g, k, 0)),
        interpret=_INTERP,
    )(d3, w1row, b1row, w2, b2)


def _kedge_body(e_ref, ghr_ref, hs_ref, ecg_ref, we, w2, b2, u_ref, en_ref):
    e = e_ref[0]
    tmp = (jnp.dot(e, we[...], preferred_element_type=jnp.float32)
           + hs_ref[0] + ghr_ref[0] + ecg_ref[0])
    u = jnp.dot(jax.nn.gelu(tmp), w2[...], preferred_element_type=jnp.float32) + b2[...]
    u_ref[0] = u
    en_ref[0] = e + u


def _kedge(e, ghr, hs, ecg3, we, w2, b2):
    out_shape = (jax.ShapeDtypeStruct((B, E, LAT), jnp.float32),
                 jax.ShapeDtypeStruct((B, E, LAT), jnp.float32))
    return pl.pallas_call(
        _kedge_body,
        grid=(B, K),
        out_shape=out_shape,
        in_specs=[pl.BlockSpec((1, N, LAT), lambda g, k: (g, k, 0)),
                  pl.BlockSpec((1, N, LAT), lambda g, k: (g, k, 0)),
                  pl.BlockSpec((1, N, LAT), lambda g, k: (g, 0, 0)),
                  pl.BlockSpec((1, 1, LAT), lambda g, k: (g, 0, 0)),
                  _fs(we.shape), _fs(w2.shape), _fs(b2.shape)],
        out_specs=(pl.BlockSpec((1, N, LAT), lambda g, k: (g, k, 0)),
                   pl.BlockSpec((1, N, LAT), lambda g, k: (g, k, 0))),
        interpret=_INTERP,
    )(e, ghr, hs, ecg3, we, w2, b2)


def _knode_body(h_ref, agg_ref, ncg_ref, a1, a2, w2, b2, wsn, wrn,
                h_out, hs_out, hr_out):
    h = h_ref[0]
    agg = agg_ref[0, 0] + agg_ref[1, 0]
    tmp = (jnp.dot(h, a1[...], preferred_element_type=jnp.float32)
           + jnp.dot(agg, a2[...], preferred_element_type=jnp.float32)
           + ncg_ref[0])
    hn = h + jnp.dot(jax.nn.gelu(tmp), w2[...], preferred_element_type=jnp.float32) + b2[...]
    h_out[0] = hn
    hs_out[0] = jnp.dot(hn, wsn[...], preferred_element_type=jnp.float32)
    hr_out[0] = jnp.dot(hn, wrn[...], preferred_element_type=jnp.float32)


def _knode(h, aggp, ncg3, a1, a2, w2, b2, wsn, wrn):
    out_shape = tuple(jax.ShapeDtypeStruct((B, N, LAT), jnp.float32) for _ in range(3))
    return pl.pallas_call(
        _knode_body,
        grid=(B,),
        out_shape=out_shape,
        in_specs=[pl.BlockSpec((1, N, LAT), lambda g: (g, 0, 0)),
                  pl.BlockSpec((2, 1, N, LAT), lambda g: (0, g, 0, 0)),
                  pl.BlockSpec((1, 1, LAT), lambda g: (g, 0, 0)),
                  _fs(a1.shape), _fs(a2.shape), _fs(w2.shape), _fs(b2.shape),
                  _fs(wsn.shape), _fs(wrn.shape)],
        out_specs=tuple(pl.BlockSpec((1, N, LAT), lambda g: (g, 0, 0)) for _ in range(3)),
        interpret=_INTERP,
    )(h, aggp, ncg3, a1, a2, w2, b2, wsn, wrn)


def _knode_final_body(z_ref, h_ref, agg_ref, ncg_ref, a1, a2, w2, b2,
                      d1, db1, d2, db2, eps_ref):
    h = h_ref[0]
    agg = agg_ref[0, 0] + agg_ref[1, 0]
    tmp = (jnp.dot(h, a1[...], preferred_element_type=jnp.float32)
           + jnp.dot(agg, a2[...], preferred_element_type=jnp.float32)
           + ncg_ref[0])
    hn = h + jnp.dot(jax.nn.gelu(tmp), w2[...], preferred_element_type=jnp.float32) + b2[...]
    dec = jax.nn.gelu(jnp.dot(hn, d1[...], preferred_element_type=jnp.float32) + db1[...])
    dec = jnp.dot(dec, d2[...], preferred_element_type=jnp.float32) + db2[...]
    eps_ref[0] = z_ref[0] - dec


def _knode_final(z, h, aggp, ncg3, a1, a2, w2, b2, d1, db1, d2, db2):
    return pl.pallas_call(
        _knode_final_body,
        grid=(B,),
        out_shape=jax.ShapeDtypeStruct((B, N, D), jnp.float32),
        in_specs=[pl.BlockSpec((1, N, D), lambda g: (g, 0, 0)),
                  pl.BlockSpec((1, N, LAT), lambda g: (g, 0, 0)),
                  pl.BlockSpec((2, 1, N, LAT), lambda g: (0, g, 0, 0)),
                  pl.BlockSpec((1, 1, LAT), lambda g: (g, 0, 0)),
                  _fs(a1.shape), _fs(a2.shape), _fs(w2.shape), _fs(b2.shape),
                  _fs(d1.shape), _fs(db1.shape), _fs(d2.shape), _fs(db2.shape)],
        out_specs=pl.BlockSpec((1, N, D), lambda g: (g, 0, 0)),
        interpret=_INTERP,
    )(z, h, aggp, ncg3, a1, a2, w2, b2, d1, db1, d2, db2)


def _sc_gather(table, idx):
    mesh = plsc.VectorSubcoreMesh(core_axis_name="c", subcore_axis_name="s")

    @functools.partial(
        pl.kernel, mesh=mesh,
        out_type=jax.ShapeDtypeStruct((BE, LAT), jnp.float32),
        scratch_types=[pltpu.VMEM((CH,), jnp.int32),
                       pltpu.VMEM((CH, LAT), jnp.float32),
                       pltpu.SemaphoreType.DMA],
    )
    def gather_k(table_hbm, idx_hbm, out_hbm, idx_v, rows_v, sem):
        wid = lax.axis_index("s") * 2 + lax.axis_index("c")
        base = wid * PER_W

        @pl.loop(0, PER_W, step=CH)
        def _(off):
            pltpu.sync_copy(idx_hbm.at[pl.ds(base + off, CH)], idx_v)
            pltpu.async_copy(table_hbm.at[idx_v], rows_v, sem).wait()
            pltpu.sync_copy(rows_v, out_hbm.at[pl.ds(base + off, CH)])

    return gather_k(table, idx)


def _sc_scatter(u, idx):
    mesh = plsc.VectorSubcoreMesh(core_axis_name="c", subcore_axis_name="s")
    ZR = BN // 16

    @functools.partial(
        pl.kernel, mesh=mesh,
        out_type=jax.ShapeDtypeStruct((2, BN, LAT), jnp.float32),
        scratch_types=[pltpu.VMEM((CH,), jnp.int32),
                       pltpu.VMEM((CH, LAT), jnp.float32),
                       pltpu.VMEM((ZR, LAT), jnp.float32),
                       pltpu.VMEM_SHARED((BN, LAT), jnp.float32),
                       pltpu.SemaphoreType.DMA],
    )
    def scatter_k(u_hbm, idx_hbm, out_hbm, idx_v, rows_v, zero_v, acc_sh, sem):
        c = lax.axis_index("c")
        s = lax.axis_index("s")

        @pl.loop(0, ZR)
        def _(r):
            for j in range(LAT // 16):
                zero_v[r, pl.ds(16 * j, 16)] = jnp.zeros((16,), jnp.float32)

        pltpu.sync_copy(zero_v, acc_sh.at[pl.ds(s * ZR, ZR)])
        plsc.subcore_barrier()

        base = (c * 16 + s) * PER_W

        @pl.loop(0, PER_W, step=CH)
        def _(off):
            pltpu.sync_copy(idx_hbm.at[pl.ds(base + off, CH)], idx_v)
            pltpu.sync_copy(u_hbm.at[pl.ds(base + off, CH)], rows_v)
            pltpu.sync_copy(rows_v, acc_sh.at[idx_v], add=True)

        plsc.subcore_barrier()
        pltpu.sync_copy(acc_sh.at[pl.ds(s * ZR, ZR)], out_hbm.at[c, pl.ds(s * ZR, ZR)])

    return scatter_k(u, idx)


def kernel(z, t, conditioning, mask, params):
    z = z.astype(jnp.float32)
    mp = params["mp"]
    We = [mp[s]["edge"][0]["W"][0:LAT] for s in range(3)]
    Ws = [mp[s]["edge"][0]["W"][LAT:2 * LAT] for s in range(3)]
    Wr = [mp[s]["edge"][0]["W"][2 * LAT:3 * LAT] for s in range(3)]
    Wg = jnp.stack([mp[s]["edge"][0]["W"][3 * LAT:] for s in range(3)])
    beg = jnp.stack([mp[s]["edge"][0]["b"].reshape(1, LAT) for s in range(3)])
    W2e = [mp[s]["edge"][1]["W"] for s in range(3)]
    b2e = [mp[s]["edge"][1]["b"].reshape(1, LAT) for s in range(3)]
    A1 = [mp[s]["node"][0]["W"][0:LAT] for s in range(3)]
    A2 = [mp[s]["node"][0]["W"][LAT:2 * LAT] for s in range(3)]
    Ag = jnp.stack([mp[s]["node"][0]["W"][2 * LAT:] for s in range(3)])
    bng = jnp.stack([mp[s]["node"][0]["b"].reshape(1, LAT) for s in range(3)])
    W2n = [mp[s]["node"][1]["W"] for s in range(3)]
    b2n = [mp[s]["node"][1]["b"].reshape(1, LAT) for s in range(3)]
    cw = [(l["W"], l["b"].reshape(1, -1)) for l in params["cond_mlp"]]
    new = [(l["W"], l["b"].reshape(1, -1)) for l in params["node_enc"]]
    eew = params["edge_enc"]
    ee_w1row = eew[0]["W"].reshape(1, LAT)
    ee_b1 = eew[0]["b"].reshape(1, LAT)
    ee_w2 = eew[1]["W"]
    ee_b2 = eew[1]["b"].reshape(1, LAT)
    d1 = params["node_dec"][0]["W"]
    db1 = params["node_dec"][0]["b"].reshape(1, LAT)
    d2 = params["node_dec"][1]["W"]
    db2 = params["node_dec"][1]["b"].reshape(1, D)

    half = D_T // 2
    freqs = jnp.asarray(
        np.exp(-np.log(10000.0) * np.arange(half, dtype=np.float32) / (half - 1))
    ).reshape(1, half)

    t_all = (t * jnp.ones((B,), jnp.float32)).reshape(B, 1)
    ecg, ncg = _k0(t_all, conditioning, freqs, cw, Wg, beg, Ag, bng)

    idx, de, h, hs, hr = _k1a(z, mask.reshape(B, 1, N), new, Ws[0], Wr[0])
    rcv = idx.reshape(BE)
    e = _k1b(de.reshape(B, E // CH, CH), ee_w1row, ee_b1, ee_w2, ee_b2)

    eps = None
    for s in range(3):
        ghr = _sc_gather(hr.reshape(BN, LAT), rcv).reshape(B, E, LAT)
        u, e = _kedge(e, ghr, hs, ecg[s].reshape(B, 1, LAT), We[s], W2e[s], b2e[s])
        aggp = _sc_scatter(u.reshape(BE, LAT), rcv).reshape(2, B, N, LAT)
        if s < 2:
            h, hs, hr = _knode(h, aggp, ncg[s].reshape(B, 1, LAT),
                               A1[s], A2[s], W2n[s], b2n[s], Ws[s + 1], Wr[s + 1])
        else:
            eps = _knode_final(z, h, aggp, ncg[s].reshape(B, 1, LAT),
                               A1[s], A2[s], W2n[s], b2n[s], d1, db1, d2, db2)
    return eps


# speedup vs baseline: 12.5026x; 1.1566x over previous
import functools

import jax
import jax.numpy as jnp
import numpy as np
from jax import lax
from jax.experimental import pallas as pl
from jax.experimental.pallas import tpu as pltpu
from jax.experimental.pallas import tpu_sc as plsc

B, N, D = 4, 1024, 3
K = 20
LAT = 128
D_T = 32
D_COND = 40
E = N * K
BN = B * N
BE = B * E
NW = 32
PER_W = BE // NW
CH = 128
_INTERP = False


def _fs(shape):
    return pl.BlockSpec(shape, lambda *_: (0,) * len(shape))


def _k0_body(t_ref, cond_ref, freqs_ref, w0, b0, w1, b1, w2, b2,
             wg, beg, ag, bng, ecg_ref, ncg_ref):
    t = t_ref[...]
    args = t * freqs_ref[...]
    x = jnp.concatenate([jnp.sin(args), jnp.cos(args), cond_ref[...]], axis=1)
    x = jax.nn.gelu(jnp.dot(x, w0[...], preferred_element_type=jnp.float32) + b0[...])
    x = jax.nn.gelu(jnp.dot(x, w1[...], preferred_element_type=jnp.float32) + b1[...])
    g = jnp.dot(x, w2[...], preferred_element_type=jnp.float32) + b2[...]
    for s in range(3):
        ecg_ref[s] = jnp.dot(g, wg[s], preferred_element_type=jnp.float32) + beg[s]
        ncg_ref[s] = jnp.dot(g, ag[s], preferred_element_type=jnp.float32) + bng[s]


def _k0(t2, cond, freqs, cw, wg, beg, ag, bng):
    (w0, b0), (w1, b1), (w2, b2) = cw
    out_shape = (jax.ShapeDtypeStruct((3, B, LAT), jnp.float32),
                 jax.ShapeDtypeStruct((3, B, LAT), jnp.float32))
    args = (t2, cond, freqs, w0, b0, w1, b1, w2, b2, wg, beg, ag, bng)
    return pl.pallas_call(
        _k0_body,
        out_shape=out_shape,
        in_specs=[_fs(a.shape) for a in args],
        out_specs=(_fs((3, B, LAT)), _fs((3, B, LAT))),
        interpret=_INTERP,
    )(*args)


def _k1a_body(z_ref, m_ref, n0w, n0b, n1w, n1b, ws0, wr0,
              idx_ref, de_ref, h_ref, hs_ref, hr_ref):
    g = pl.program_id(0)
    x = z_ref[0]
    sqc = jnp.sum(x * x, axis=1, keepdims=True)
    sqr = jnp.transpose(sqc)
    xx = lax.dot_general(x, x, (((1,), (1,)), ((), ())),
                         preferred_element_type=jnp.float32)
    dist = sqc + sqr - 2.0 * xx
    m_col = jnp.transpose(m_ref[0])
    dist = jnp.where(m_col > 0, dist, 1e10)
    s1 = jnp.dot(x, jnp.ones((3, 1), jnp.float32),
                 preferred_element_type=jnp.float32)
    s1_row = jnp.transpose(s1)
    sub = lax.broadcasted_iota(jnp.int32, (N, N), 0)
    big_i = jnp.int32(2 ** 30)
    for k in range(K):
        mn = jnp.min(dist, axis=0, keepdims=True)
        amin = jnp.min(jnp.where(dist == mn, sub, big_i), axis=0,
                       keepdims=True)
        sel = sub == amin
        tgt_s = jnp.sum(jnp.where(sel, s1, 0.0), axis=0, keepdims=True)
        idx_ref[0, k] = (amin + g * N)[0]
        de_ref[0, k] = (s1_row - tgt_s)[0]
        dist = jnp.where(sel, 1e10, dist)
    h = jax.nn.gelu(jnp.dot(x, n0w[...], preferred_element_type=jnp.float32) + n0b[...])
    h = jnp.dot(h, n1w[...], preferred_element_type=jnp.float32) + n1b[...]
    h_ref[0] = h
    hs_ref[0] = jnp.dot(h, ws0[...], preferred_element_type=jnp.float32)
    hr_ref[0] = jnp.dot(h, wr0[...], preferred_element_type=jnp.float32)


def _k1a(z, mask3, new, ws0, wr0):
    (n0w, n0b), (n1w, n1b) = new
    out_shape = (jax.ShapeDtypeStruct((B, K, N), jnp.int32),
                 jax.ShapeDtypeStruct((B, K, N), jnp.float32),
                 jax.ShapeDtypeStruct((B, N, LAT), jnp.float32),
                 jax.ShapeDtypeStruct((B, N, LAT), jnp.float32),
                 jax.ShapeDtypeStruct((B, N, LAT), jnp.float32))
    wspecs = [_fs(w.shape) for w in (n0w, n0b, n1w, n1b, ws0, wr0)]
    return pl.pallas_call(
        _k1a_body,
        grid=(B,),
        out_shape=out_shape,
        in_specs=[pl.BlockSpec((1, N, D), lambda g: (g, 0, 0)),
                  pl.BlockSpec((1, 1, N), lambda g: (g, 0, 0))] + wspecs,
        out_specs=(pl.BlockSpec((1, K, N), lambda g: (g, 0, 0)),
                   pl.BlockSpec((1, K, N), lambda g: (g, 0, 0)),
                   pl.BlockSpec((1, N, LAT), lambda g: (g, 0, 0)),
                   pl.BlockSpec((1, N, LAT), lambda g: (g, 0, 0)),
                   pl.BlockSpec((1, N, LAT), lambda g: (g, 0, 0))),
        interpret=_INTERP,
    )(z, mask3, n0w, n0b, n1w, n1b, ws0, wr0)


def _k1b_body(d_ref, w1row, b1row, w2, b2, e_ref):
    cols = []
    for r in range(8):
        cols.append(jnp.transpose(d_ref[0, r:r + 1, :]))
    v = jnp.concatenate(cols, axis=0)
    a = jax.nn.gelu(v * w1row[...] + b1row[...])
    e_ref[0] = jnp.dot(a, w2[...], preferred_element_type=jnp.float32) + b2[...]


def _k1b(d3, w1row, b1row, w2, b2):
    return pl.pallas_call(
        _k1b_body,
        grid=(B, K),
        out_shape=jax.ShapeDtypeStruct((B, E, LAT), jnp.float32),
        in_specs=[pl.BlockSpec((1, 8, CH), lambda g, k: (g, k, 0)),
                  _fs(w1row.shape), _fs(b1row.shape), _fs(w2.shape), _fs(b2.shape)],
        out_specs=pl.BlockSpec((1, N, LAT), lambda g, k: (g, k, 0)),
        interpret=_INTERP,
    )(d3, w1row, b1row, w2, b2)


def _kedge_body(e_ref, ghr_ref, hs_ref, ecg_ref, we, w2, b2, u_ref, en_ref):
    e = e_ref[0]
    tmp = (jnp.dot(e, we[...], preferred_element_type=jnp.float32)
           + hs_ref[0] + ghr_ref[0] + ecg_ref[0])
    u = jnp.dot(jax.nn.gelu(tmp), w2[...], preferred_element_type=jnp.float32) + b2[...]
    u_ref[0] = u
    en_ref[0] = e + u


def _kedge(e, ghr, hs, ecg3, we, w2, b2):
    out_shape = (jax.ShapeDtypeStruct((B, E, LAT), jnp.float32),
                 jax.ShapeDtypeStruct((B, E, LAT), jnp.float32))
    return pl.pallas_call(
        _kedge_body,
        grid=(B, K),
        out_shape=out_shape,
        in_specs=[pl.BlockSpec((1, N, LAT), lambda g, k: (g, k, 0)),
                  pl.BlockSpec((1, N, LAT), lambda g, k: (g, k, 0)),
                  pl.BlockSpec((1, N, LAT), lambda g, k: (g, 0, 0)),
                  pl.BlockSpec((1, 1, LAT), lambda g, k: (g, 0, 0)),
                  _fs(we.shape), _fs(w2.shape), _fs(b2.shape)],
        out_specs=(pl.BlockSpec((1, N, LAT), lambda g, k: (g, k, 0)),
                   pl.BlockSpec((1, N, LAT), lambda g, k: (g, k, 0))),
        interpret=_INTERP,
    )(e, ghr, hs, ecg3, we, w2, b2)


def _knode_body(h_ref, agg_ref, ncg_ref, a1, a2, w2, b2, wsn, wrn,
                h_out, hs_out, hr_out):
    h = h_ref[0]
    agg = agg_ref[0, 0] + agg_ref[1, 0]
    tmp = (jnp.dot(h, a1[...], preferred_element_type=jnp.float32)
           + jnp.dot(agg, a2[...], preferred_element_type=jnp.float32)
           + ncg_ref[0])
    hn = h + jnp.dot(jax.nn.gelu(tmp), w2[...], preferred_element_type=jnp.float32) + b2[...]
    h_out[0] = hn
    hs_out[0] = jnp.dot(hn, wsn[...], preferred_element_type=jnp.float32)
    hr_out[0] = jnp.dot(hn, wrn[...], preferred_element_type=jnp.float32)


def _knode(h, aggp, ncg3, a1, a2, w2, b2, wsn, wrn):
    out_shape = tuple(jax.ShapeDtypeStruct((B, N, LAT), jnp.float32) for _ in range(3))
    return pl.pallas_call(
        _knode_body,
        grid=(B,),
        out_shape=out_shape,
        in_specs=[pl.BlockSpec((1, N, LAT), lambda g: (g, 0, 0)),
                  pl.BlockSpec((2, 1, N, LAT), lambda g: (0, g, 0, 0)),
                  pl.BlockSpec((1, 1, LAT), lambda g: (g, 0, 0)),
                  _fs(a1.shape), _fs(a2.shape), _fs(w2.shape), _fs(b2.shape),
                  _fs(wsn.shape), _fs(wrn.shape)],
        out_specs=tuple(pl.BlockSpec((1, N, LAT), lambda g: (g, 0, 0)) for _ in range(3)),
        interpret=_INTERP,
    )(h, aggp, ncg3, a1, a2, w2, b2, wsn, wrn)


def _knode_final_body(z_ref, h_ref, agg_ref, ncg_ref, a1, a2, w2, b2,
                      d1, db1, d2, db2, eps_ref):
    h = h_ref[0]
    agg = agg_ref[0, 0] + agg_ref[1, 0]
    tmp = (jnp.dot(h, a1[...], preferred_element_type=jnp.float32)
           + jnp.dot(agg, a2[...], preferred_element_type=jnp.float32)
           + ncg_ref[0])
    hn = h + jnp.dot(jax.nn.gelu(tmp), w2[...], preferred_element_type=jnp.float32) + b2[...]
    dec = jax.nn.gelu(jnp.dot(hn, d1[...], preferred_element_type=jnp.float32) + db1[...])
    dec = jnp.dot(dec, d2[...], preferred_element_type=jnp.float32) + db2[...]
    eps_ref[0] = z_ref[0] - dec


def _knode_final(z, h, aggp, ncg3, a1, a2, w2, b2, d1, db1, d2, db2):
    return pl.pallas_call(
        _knode_final_body,
        grid=(B,),
        out_shape=jax.ShapeDtypeStruct((B, N, D), jnp.float32),
        in_specs=[pl.BlockSpec((1, N, D), lambda g: (g, 0, 0)),
                  pl.BlockSpec((1, N, LAT), lambda g: (g, 0, 0)),
                  pl.BlockSpec((2, 1, N, LAT), lambda g: (0, g, 0, 0)),
                  pl.BlockSpec((1, 1, LAT), lambda g: (g, 0, 0)),
                  _fs(a1.shape), _fs(a2.shape), _fs(w2.shape), _fs(b2.shape),
                  _fs(d1.shape), _fs(db1.shape), _fs(d2.shape), _fs(db2.shape)],
        out_specs=pl.BlockSpec((1, N, D), lambda g: (g, 0, 0)),
        interpret=_INTERP,
    )(z, h, aggp, ncg3, a1, a2, w2, b2, d1, db1, d2, db2)


NCHUNK = PER_W // CH
NBUF = 4


def _sc_gather(table, idx2):
    mesh = plsc.VectorSubcoreMesh(core_axis_name="c", subcore_axis_name="s")

    @functools.partial(
        pl.kernel, mesh=mesh,
        out_type=jax.ShapeDtypeStruct((BE, LAT), jnp.float32),
        scratch_types=[pltpu.VMEM((NCHUNK, CH), jnp.int32)]
        + [pltpu.VMEM((CH, LAT), jnp.float32) for _ in range(NBUF)]
        + [pltpu.SemaphoreType.DMA for _ in range(2 * NBUF)],
    )
    def gather_k(table_hbm, idx_hbm, out_hbm, idx_v, *rest):
        rows = rest[:NBUF]
        semg = rest[NBUF:2 * NBUF]
        semw = rest[2 * NBUF:]
        wid = lax.axis_index("s") * 2 + lax.axis_index("c")
        base = wid * PER_W
        pltpu.sync_copy(idx_hbm.at[wid], idx_v)
        hg, hw = {}, {}

        def start_wb(d):
            hg[d].wait()
            hw[d] = pltpu.async_copy(
                rows[d % NBUF], out_hbm.at[pl.ds(base + d * CH, CH)], semw[d % NBUF])

        for c in range(NCHUNK):
            if c >= NBUF:
                hw[c - NBUF].wait()
            hg[c] = pltpu.async_copy(table_hbm.at[idx_v.at[c]], rows[c % NBUF],
                                     semg[c % NBUF])
            if c - (NBUF - 1) >= 0:
                start_wb(c - (NBUF - 1))
        for d in range(NCHUNK - NBUF + 1, NCHUNK):
            start_wb(d)
        for d in range(NCHUNK - NBUF, NCHUNK):
            hw[d].wait()

    return gather_k(table, idx2)


def _sc_scatter(u, idx2):
    mesh = plsc.VectorSubcoreMesh(core_axis_name="c", subcore_axis_name="s")
    ZR = BN // 16

    @functools.partial(
        pl.kernel, mesh=mesh,
        out_type=jax.ShapeDtypeStruct((2, BN, LAT), jnp.float32),
        scratch_types=[pltpu.VMEM((NCHUNK, CH), jnp.int32),
                       pltpu.VMEM_SHARED((BN, LAT), jnp.float32)]
        + [pltpu.VMEM((CH, LAT), jnp.float32) for _ in range(NBUF)]
        + [pltpu.SemaphoreType.DMA for _ in range(2 * NBUF)],
    )
    def scatter_k(u_hbm, idx_hbm, out_hbm, idx_v, acc_sh, *rest):
        rows = rest[:NBUF]
        seml = rest[NBUF:2 * NBUF]
        sema = rest[2 * NBUF:]
        cc = lax.axis_index("c")
        s = lax.axis_index("s")

        @pl.loop(0, CH)
        def _(r):
            for j in range(LAT // 16):
                rows[0][r, pl.ds(16 * j, 16)] = jnp.zeros((16,), jnp.float32)

        for half in range(ZR // CH):
            pltpu.sync_copy(rows[0], acc_sh.at[pl.ds(s * ZR + half * CH, CH)])
        plsc.subcore_barrier()

        base = (cc * 16 + s) * PER_W
        pltpu.sync_copy(idx_hbm.at[cc * 16 + s], idx_v)
        hl, ha = {}, {}

        def start_add(d):
            hl[d].wait()
            ha[d] = pltpu.async_copy(rows[d % NBUF], acc_sh.at[idx_v.at[d]],
                                     sema[d % NBUF], add=True)

        for c in range(NCHUNK):
            if c >= NBUF:
                ha[c - NBUF].wait()
            hl[c] = pltpu.async_copy(u_hbm.at[pl.ds(base + c * CH, CH)],
                                     rows[c % NBUF], seml[c % NBUF])
            if c - (NBUF - 1) >= 0:
                start_add(c - (NBUF - 1))
        for d in range(NCHUNK - NBUF + 1, NCHUNK):
            start_add(d)
        for d in range(NCHUNK - NBUF, NCHUNK):
            ha[d].wait()

        plsc.subcore_barrier()
        pltpu.sync_copy(acc_sh.at[pl.ds(s * ZR, ZR)], out_hbm.at[cc, pl.ds(s * ZR, ZR)])

    return scatter_k(u, idx2)


def kernel(z, t, conditioning, mask, params):
    z = z.astype(jnp.float32)
    mp = params["mp"]
    We = [mp[s]["edge"][0]["W"][0:LAT] for s in range(3)]
    Ws = [mp[s]["edge"][0]["W"][LAT:2 * LAT] for s in range(3)]
    Wr = [mp[s]["edge"][0]["W"][2 * LAT:3 * LAT] for s in range(3)]
    Wg = jnp.stack([mp[s]["edge"][0]["W"][3 * LAT:] for s in range(3)])
    beg = jnp.stack([mp[s]["edge"][0]["b"].reshape(1, LAT) for s in range(3)])
    W2e = [mp[s]["edge"][1]["W"] for s in range(3)]
    b2e = [mp[s]["edge"][1]["b"].reshape(1, LAT) for s in range(3)]
    A1 = [mp[s]["node"][0]["W"][0:LAT] for s in range(3)]
    A2 = [mp[s]["node"][0]["W"][LAT:2 * LAT] for s in range(3)]
    Ag = jnp.stack([mp[s]["node"][0]["W"][2 * LAT:] for s in range(3)])
    bng = jnp.stack([mp[s]["node"][0]["b"].reshape(1, LAT) for s in range(3)])
    W2n = [mp[s]["node"][1]["W"] for s in range(3)]
    b2n = [mp[s]["node"][1]["b"].reshape(1, LAT) for s in range(3)]
    cw = [(l["W"], l["b"].reshape(1, -1)) for l in params["cond_mlp"]]
    new = [(l["W"], l["b"].reshape(1, -1)) for l in params["node_enc"]]
    eew = params["edge_enc"]
    ee_w1row = eew[0]["W"].reshape(1, LAT)
    ee_b1 = eew[0]["b"].reshape(1, LAT)
    ee_w2 = eew[1]["W"]
    ee_b2 = eew[1]["b"].reshape(1, LAT)
    d1 = params["node_dec"][0]["W"]
    db1 = params["node_dec"][0]["b"].reshape(1, LAT)
    d2 = params["node_dec"][1]["W"]
    db2 = params["node_dec"][1]["b"].reshape(1, D)

    half = D_T // 2
    freqs = jnp.asarray(
        np.exp(-np.log(10000.0) * np.arange(half, dtype=np.float32) / (half - 1))
    ).reshape(1, half)

    t_all = (t * jnp.ones((B,), jnp.float32)).reshape(B, 1)
    ecg, ncg = _k0(t_all, conditioning, freqs, cw, Wg, beg, Ag, bng)

    idx, de, h, hs, hr = _k1a(z, mask.reshape(B, 1, N), new, Ws[0], Wr[0])
    rcv2 = idx.reshape(NW, NCHUNK, CH)
    e = _k1b(de.reshape(B, E // CH, CH), ee_w1row, ee_b1, ee_w2, ee_b2)

    eps = None
    for s in range(3):
        ghr = _sc_gather(hr.reshape(BN, LAT), rcv2).reshape(B, E, LAT)
        u, e = _kedge(e, ghr, hs, ecg[s].reshape(B, 1, LAT), We[s], W2e[s], b2e[s])
        aggp = _sc_scatter(u.reshape(BE, LAT), rcv2).reshape(2, B, N, LAT)
        if s < 2:
            h, hs, hr = _knode(h, aggp, ncg[s].reshape(B, 1, LAT),
                               A1[s], A2[s], W2n[s], b2n[s], Ws[s + 1], Wr[s + 1])
        else:
            eps = _knode_final(z, h, aggp, ncg[s].reshape(B, 1, LAT),
                               A1[s], A2[s], W2n[s], b2n[s], d1, db1, d2, db2)
    return eps


# speedup vs baseline: 13.7156x; 1.0970x over previous
import functools

import jax
import jax.numpy as jnp
import numpy as np
from jax import lax
from jax.experimental import pallas as pl
from jax.experimental.pallas import tpu as pltpu
from jax.experimental.pallas import tpu_sc as plsc

B, N, D = 4, 1024, 3
K = 20
LAT = 128
D_T = 32
HALF = 2
E = N * K
E2 = HALF * E
N2 = HALF * N
NW = 32
PER_W = E2 // NW
CH = 128
NCHUNK = PER_W // CH
NBUF = 4


def _fs(shape):
    return pl.BlockSpec(shape, lambda *_: (0,) * len(shape))


def _k1_body(z_ref, m_ref, t_ref, c_ref, freqs,
             cw0, cb0, cw1, cb1, cw2, cb2, wg, beg, ag, bng,
             n0w, n0b, n1w, n1b, ws0, wr0,
             idx_ref, de_ref, h_ref, hs_ref, hr_ref, ecg_ref, ncg_ref):
    g = pl.program_id(0)
    args = t_ref[0] * freqs[...]
    cvec = jnp.concatenate([jnp.sin(args), jnp.cos(args), c_ref[0]], axis=1)
    cvec = jax.nn.gelu(jnp.dot(cvec, cw0[...], preferred_element_type=jnp.float32) + cb0[...])
    cvec = jax.nn.gelu(jnp.dot(cvec, cw1[...], preferred_element_type=jnp.float32) + cb1[...])
    cvec = jnp.dot(cvec, cw2[...], preferred_element_type=jnp.float32) + cb2[...]
    ecg_ref[0] = jnp.concatenate(
        [jnp.dot(cvec, wg[s], preferred_element_type=jnp.float32) + beg[s]
         for s in range(3)], axis=0)
    ncg_ref[0] = jnp.concatenate(
        [jnp.dot(cvec, ag[s], preferred_element_type=jnp.float32) + bng[s]
         for s in range(3)], axis=0)

    x = z_ref[0]
    sqc = jnp.sum(x * x, axis=1, keepdims=True)
    sqr = jnp.transpose(sqc)
    xx = lax.dot_general(x, x, (((1,), (1,)), ((), ())),
                         preferred_element_type=jnp.float32)
    dist = sqc + sqr - 2.0 * xx
    m_col = jnp.transpose(m_ref[0])
    dist = jnp.where(m_col > 0, dist, 1e10)
    s1 = jnp.dot(x, jnp.ones((3, 1), jnp.float32),
                 preferred_element_type=jnp.float32)
    s1_row = jnp.transpose(s1)
    sub = lax.broadcasted_iota(jnp.int32, (N, N), 0)
    big_i = jnp.int32(2 ** 30)
    half_off = lax.rem(g, 2) * N
    for k in range(K):
        mn = jnp.min(dist, axis=0, keepdims=True)
        amin = jnp.min(jnp.where(dist == mn, sub, big_i), axis=0,
                       keepdims=True)
        sel = sub == amin
        tgt_s = jnp.sum(jnp.where(sel, s1, 0.0), axis=0, keepdims=True)
        idx_ref[0, k] = (amin + half_off)[0]
        de_ref[0, k] = (s1_row - tgt_s)[0]
        dist = jnp.where(sel, 1e10, dist)

    h = jax.nn.gelu(jnp.dot(x, n0w[...], preferred_element_type=jnp.float32) + n0b[...])
    h = jnp.dot(h, n1w[...], preferred_element_type=jnp.float32) + n1b[...]
    h_ref[0] = h
    hs_ref[0] = jnp.dot(h, ws0[...], preferred_element_type=jnp.float32)
    hr_ref[0] = jnp.dot(h, wr0[...], preferred_element_type=jnp.float32)


def _k1(z, mask3, t3, cond3, freqs, cw, wg, beg, ag, bng, new, ws0, wr0):
    (cw0, cb0), (cw1, cb1), (cw2, cb2) = cw
    (n0w, n0b), (n1w, n1b) = new
    out_shape = (jax.ShapeDtypeStruct((B, K, N), jnp.int32),
                 jax.ShapeDtypeStruct((B, K, N), jnp.float32),
                 jax.ShapeDtypeStruct((B, N, LAT), jnp.float32),
                 jax.ShapeDtypeStruct((B, N, LAT), jnp.float32),
                 jax.ShapeDtypeStruct((B, N, LAT), jnp.float32),
                 jax.ShapeDtypeStruct((B, 3, LAT), jnp.float32),
                 jax.ShapeDtypeStruct((B, 3, LAT), jnp.float32))
    ws = (freqs, cw0, cb0, cw1, cb1, cw2, cb2, wg, beg, ag, bng,
          n0w, n0b, n1w, n1b, ws0, wr0)
    return pl.pallas_call(
        _k1_body,
        grid=(B,),
        out_shape=out_shape,
        in_specs=[pl.BlockSpec((1, N, D), lambda g: (g, 0, 0)),
                  pl.BlockSpec((1, 1, N), lambda g: (g, 0, 0)),
                  pl.BlockSpec((1, 1, 1), lambda g: (g, 0, 0)),
                  pl.BlockSpec((1, 1, 8), lambda g: (g, 0, 0))]
        + [_fs(w.shape) for w in ws],
        out_specs=(pl.BlockSpec((1, K, N), lambda g: (g, 0, 0)),
                   pl.BlockSpec((1, K, N), lambda g: (g, 0, 0)),
                   pl.BlockSpec((1, N, LAT), lambda g: (g, 0, 0)),
                   pl.BlockSpec((1, N, LAT), lambda g: (g, 0, 0)),
                   pl.BlockSpec((1, N, LAT), lambda g: (g, 0, 0)),
                   pl.BlockSpec((1, 3, LAT), lambda g: (g, 0, 0)),
                   pl.BlockSpec((1, 3, LAT), lambda g: (g, 0, 0))),
    )(z, mask3, t3, cond3, *ws)


def _edge_common(e, ghr, hs, ecg, we, w2, b2, u_ref, en_ref):
    tmp = (jnp.dot(e, we[...], preferred_element_type=jnp.float32)
           + hs + ghr + ecg)
    u = jnp.dot(jax.nn.gelu(tmp), w2[...], preferred_element_type=jnp.float32) + b2[...]
    u_ref[0] = u
    en_ref[0] = e + u


def _kedge_body(e_ref, ghr_ref, hs_ref, ecg_ref, we, w2, b2, u_ref, en_ref):
    _edge_common(e_ref[0], ghr_ref[0], hs_ref[0], ecg_ref[0], we, w2, b2,
                 u_ref, en_ref)


def _kedge(e, ghr, hs, ecg, we, w2, b2):
    nb = e.shape[0]
    out_shape = (jax.ShapeDtypeStruct((nb, E, LAT), jnp.float32),
                 jax.ShapeDtypeStruct((nb, E, LAT), jnp.float32))
    return pl.pallas_call(
        _kedge_body,
        grid=(nb, K),
        out_shape=out_shape,
        in_specs=[pl.BlockSpec((1, N, LAT), lambda g, k: (g, k, 0)),
                  pl.BlockSpec((1, N, LAT), lambda g, k: (g, k, 0)),
                  pl.BlockSpec((1, N, LAT), lambda g, k: (g, 0, 0)),
                  pl.BlockSpec((1, 1, LAT), lambda g, k: (g, 0, 0)),
                  _fs(we.shape), _fs(w2.shape), _fs(b2.shape)],
        out_specs=(pl.BlockSpec((1, N, LAT), lambda g, k: (g, k, 0)),
                   pl.BlockSpec((1, N, LAT), lambda g, k: (g, k, 0))),
    )(e, ghr, hs, ecg, we, w2, b2)


def _kedge0_body(d_ref, ghr_ref, hs_ref, ecg_ref, ew1, eb1, ew2, eb2,
                 we, w2, b2, u_ref, en_ref):
    cols = []
    for r in range(8):
        cols.append(jnp.transpose(d_ref[0, r:r + 1, :]))
    v = jnp.concatenate(cols, axis=0)
    a = jax.nn.gelu(v * ew1[...] + eb1[...])
    e = jnp.dot(a, ew2[...], preferred_element_type=jnp.float32) + eb2[...]
    _edge_common(e, ghr_ref[0], hs_ref[0], ecg_ref[0], we, w2, b2,
                 u_ref, en_ref)


def _kedge0(d3, ghr, hs, ecg, eew, we, w2, b2):
    nb = d3.shape[0]
    ew1, eb1, ew2, eb2 = eew
    out_shape = (jax.ShapeDtypeStruct((nb, E, LAT), jnp.float32),
                 jax.ShapeDtypeStruct((nb, E, LAT), jnp.float32))
    return pl.pallas_call(
        _kedge0_body,
        grid=(nb, K),
        out_shape=out_shape,
        in_specs=[pl.BlockSpec((1, 8, CH), lambda g, k: (g, k, 0)),
                  pl.BlockSpec((1, N, LAT), lambda g, k: (g, k, 0)),
                  pl.BlockSpec((1, N, LAT), lambda g, k: (g, 0, 0)),
                  pl.BlockSpec((1, 1, LAT), lambda g, k: (g, 0, 0)),
                  _fs(ew1.shape), _fs(eb1.shape), _fs(ew2.shape), _fs(eb2.shape),
                  _fs(we.shape), _fs(w2.shape), _fs(b2.shape)],
        out_specs=(pl.BlockSpec((1, N, LAT), lambda g, k: (g, k, 0)),
                   pl.BlockSpec((1, N, LAT), lambda g, k: (g, k, 0))),
    )(d3, ghr, hs, ecg, ew1, eb1, ew2, eb2, we, w2, b2)


def _knode_body(h_ref, agg_ref, ncg_ref, a1, a2, w2, b2, wsn, wrn,
                h_out, hs_out, hr_out):
    h = h_ref[0]
    agg = agg_ref[0, 0] + agg_ref[1, 0]
    tmp = (jnp.dot(h, a1[...], preferred_element_type=jnp.float32)
           + jnp.dot(agg, a2[...], preferred_element_type=jnp.float32)
           + ncg_ref[0])
    hn = h + jnp.dot(jax.nn.gelu(tmp), w2[...], preferred_element_type=jnp.float32) + b2[...]
    h_out[0] = hn
    hs_out[0] = jnp.dot(hn, wsn[...], preferred_element_type=jnp.float32)
    hr_out[0] = jnp.dot(hn, wrn[...], preferred_element_type=jnp.float32)


def _knode(h, aggp, ncg, a1, a2, w2, b2, wsn, wrn):
    nb = h.shape[0]
    out_shape = tuple(jax.ShapeDtypeStruct((nb, N, LAT), jnp.float32) for _ in range(3))
    return pl.pallas_call(
        _knode_body,
        grid=(nb,),
        out_shape=out_shape,
        in_specs=[pl.BlockSpec((1, N, LAT), lambda g: (g, 0, 0)),
                  pl.BlockSpec((2, 1, N, LAT), lambda g: (0, g, 0, 0)),
                  pl.BlockSpec((1, 1, LAT), lambda g: (g, 0, 0)),
                  _fs(a1.shape), _fs(a2.shape), _fs(w2.shape), _fs(b2.shape),
                  _fs(wsn.shape), _fs(wrn.shape)],
        out_specs=tuple(pl.BlockSpec((1, N, LAT), lambda g: (g, 0, 0)) for _ in range(3)),
    )(h, aggp, ncg, a1, a2, w2, b2, wsn, wrn)


def _knode_final_body(z_ref, h_ref, agg_ref, ncg_ref, a1, a2, w2, b2,
                      d1, db1, d2, db2, eps_ref):
    h = h_ref[0]
    agg = agg_ref[0, 0] + agg_ref[1, 0]
    tmp = (jnp.dot(h, a1[...], preferred_element_type=jnp.float32)
           + jnp.dot(agg, a2[...], preferred_element_type=jnp.float32)
           + ncg_ref[0])
    hn = h + jnp.dot(jax.nn.gelu(tmp), w2[...], preferred_element_type=jnp.float32) + b2[...]
    dec = jax.nn.gelu(jnp.dot(hn, d1[...], preferred_element_type=jnp.float32) + db1[...])
    dec = jnp.dot(dec, d2[...], preferred_element_type=jnp.float32) + db2[...]
    eps_ref[0] = z_ref[0] - dec


def _knode_final(z, h, aggp, ncg, a1, a2, w2, b2, d1, db1, d2, db2):
    nb = h.shape[0]
    return pl.pallas_call(
        _knode_final_body,
        grid=(nb,),
        out_shape=jax.ShapeDtypeStruct((nb, N, D), jnp.float32),
        in_specs=[pl.BlockSpec((1, N, D), lambda g: (g, 0, 0)),
                  pl.BlockSpec((1, N, LAT), lambda g: (g, 0, 0)),
                  pl.BlockSpec((2, 1, N, LAT), lambda g: (0, g, 0, 0)),
                  pl.BlockSpec((1, 1, LAT), lambda g: (g, 0, 0)),
                  _fs(a1.shape), _fs(a2.shape), _fs(w2.shape), _fs(b2.shape),
                  _fs(d1.shape), _fs(db1.shape), _fs(d2.shape), _fs(db2.shape)],
        out_specs=pl.BlockSpec((1, N, D), lambda g: (g, 0, 0)),
    )(z, h, aggp, ncg, a1, a2, w2, b2, d1, db1, d2, db2)


def _sc_gather(table, idx3):
    mesh = plsc.VectorSubcoreMesh(core_axis_name="c", subcore_axis_name="s")

    @functools.partial(
        pl.kernel, mesh=mesh,
        out_type=jax.ShapeDtypeStruct((E2, LAT), jnp.float32),
        scratch_types=[pltpu.VMEM((NCHUNK, CH), jnp.int32)]
        + [pltpu.VMEM((CH, LAT), jnp.float32) for _ in range(NBUF)]
        + [pltpu.SemaphoreType.DMA for _ in range(2 * NBUF)],
    )
    def gather_k(table_hbm, idx_hbm, out_hbm, idx_v, *rest):
        rows = rest[:NBUF]
        semg = rest[NBUF:2 * NBUF]
        semw = rest[2 * NBUF:]
        wid = lax.axis_index("s") * 2 + lax.axis_index("c")
        base = wid * PER_W
        pltpu.sync_copy(idx_hbm.at[wid], idx_v)
        hg, hw = {}, {}

        def start_wb(d):
            hg[d].wait()
            hw[d] = pltpu.async_copy(
                rows[d % NBUF], out_hbm.at[pl.ds(base + d * CH, CH)], semw[d % NBUF])

        for c in range(NCHUNK):
            if c >= NBUF:
                hw[c - NBUF].wait()
            hg[c] = pltpu.async_copy(table_hbm.at[idx_v.at[c]], rows[c % NBUF],
                                     semg[c % NBUF])
            if c - (NBUF - 1) >= 0:
                start_wb(c - (NBUF - 1))
        for d in range(NCHUNK - NBUF + 1, NCHUNK):
            start_wb(d)
        for d in range(NCHUNK - NBUF, NCHUNK):
            hw[d].wait()

    return gather_k(table, idx3)


def _sc_scatter(u, idx3):
    mesh = plsc.VectorSubcoreMesh(core_axis_name="c", subcore_axis_name="s")
    ZR = N2 // 16

    @functools.partial(
        pl.kernel, mesh=mesh,
        out_type=jax.ShapeDtypeStruct((2, N2, LAT), jnp.float32),
        scratch_types=[pltpu.VMEM((NCHUNK, CH), jnp.int32),
                       pltpu.VMEM_SHARED((N2, LAT), jnp.float32)]
        + [pltpu.VMEM((CH, LAT), jnp.float32) for _ in range(NBUF)]
        + [pltpu.SemaphoreType.DMA for _ in range(2 * NBUF)],
    )
    def scatter_k(u_hbm, idx_hbm, out_hbm, idx_v, acc_sh, *rest):
        rows = rest[:NBUF]
        seml = rest[NBUF:2 * NBUF]
        sema = rest[2 * NBUF:]
        cc = lax.axis_index("c")
        s = lax.axis_index("s")

        @pl.loop(0, CH)
        def _(r):
            for j in range(LAT // 16):
                rows[0][r, pl.ds(16 * j, 16)] = jnp.zeros((16,), jnp.float32)

        for half in range(ZR // CH):
            pltpu.sync_copy(rows[0], acc_sh.at[pl.ds(s * ZR + half * CH, CH)])
        plsc.subcore_barrier()

        wid = cc * 16 + s
        base = wid * PER_W
        pltpu.sync_copy(idx_hbm.at[wid], idx_v)
        hl, ha = {}, {}

        def start_add(d):
            hl[d].wait()
            ha[d] = pltpu.async_copy(rows[d % NBUF], acc_sh.at[idx_v.at[d]],
                                     sema[d % NBUF], add=True)

        for c in range(NCHUNK):
            if c >= NBUF:
                ha[c - NBUF].wait()
            hl[c] = pltpu.async_copy(u_hbm.at[pl.ds(base + c * CH, CH)],
                                     rows[c % NBUF], seml[c % NBUF])
            if c - (NBUF - 1) >= 0:
                start_add(c - (NBUF - 1))
        for d in range(NCHUNK - NBUF + 1, NCHUNK):
            start_add(d)
        for d in range(NCHUNK - NBUF, NCHUNK):
            ha[d].wait()

        plsc.subcore_barrier()
        pltpu.sync_copy(acc_sh.at[pl.ds(s * ZR, ZR)], out_hbm.at[cc, pl.ds(s * ZR, ZR)])

    return scatter_k(u, idx3)


def kernel(z, t, conditioning, mask, params):
    z = z.astype(jnp.float32)
    mp = params["mp"]
    We = [mp[s]["edge"][0]["W"][0:LAT] for s in range(3)]
    Ws = [mp[s]["edge"][0]["W"][LAT:2 * LAT] for s in range(3)]
    Wr = [mp[s]["edge"][0]["W"][2 * LAT:3 * LAT] for s in range(3)]
    Wg = jnp.stack([mp[s]["edge"][0]["W"][3 * LAT:] for s in range(3)])
    beg = jnp.stack([mp[s]["edge"][0]["b"].reshape(1, LAT) for s in range(3)])
    W2e = [mp[s]["edge"][1]["W"] for s in range(3)]
    b2e = [mp[s]["edge"][1]["b"].reshape(1, LAT) for s in range(3)]
    A1 = [mp[s]["node"][0]["W"][0:LAT] for s in range(3)]
    A2 = [mp[s]["node"][0]["W"][LAT:2 * LAT] for s in range(3)]
    Ag = jnp.stack([mp[s]["node"][0]["W"][2 * LAT:] for s in range(3)])
    bng = jnp.stack([mp[s]["node"][0]["b"].reshape(1, LAT) for s in range(3)])
    W2n = [mp[s]["node"][1]["W"] for s in range(3)]
    b2n = [mp[s]["node"][1]["b"].reshape(1, LAT) for s in range(3)]
    cw = [(l["W"], l["b"].reshape(1, -1)) for l in params["cond_mlp"]]
    new = [(l["W"], l["b"].reshape(1, -1)) for l in params["node_enc"]]
    eew = (params["edge_enc"][0]["W"].reshape(1, LAT),
           params["edge_enc"][0]["b"].reshape(1, LAT),
           params["edge_enc"][1]["W"],
           params["edge_enc"][1]["b"].reshape(1, LAT))
    d1 = params["node_dec"][0]["W"]
    db1 = params["node_dec"][0]["b"].reshape(1, LAT)
    d2 = params["node_dec"][1]["W"]
    db2 = params["node_dec"][1]["b"].reshape(1, D)

    half_dim = D_T // 2
    freqs = jnp.asarray(
        np.exp(-np.log(10000.0) * np.arange(half_dim, dtype=np.float32) / (half_dim - 1))
    ).reshape(1, half_dim)

    t3 = (t * jnp.ones((B,), jnp.float32)).reshape(B, 1, 1)
    cond3 = conditioning.reshape(B, 1, 8)

    idx, de, h0, hs0, hr0, ecg, ncg = _k1(
        z, mask.reshape(B, 1, N), t3, cond3, freqs, cw, Wg, beg, Ag, bng,
        new, Ws[0], Wr[0])

    d3 = de.reshape(B, E // CH, CH)
    hh = [h0[0:2], h0[2:4]]
    hs = [hs0[0:2], hs0[2:4]]
    hr = [hr0[0:2], hr0[2:4]]
    rcv = [idx[0:2].reshape(NW, NCHUNK, CH), idx[2:4].reshape(NW, NCHUNK, CH)]
    d3h = [d3[0:2], d3[2:4]]
    zh = [z[0:2], z[2:4]]
    ecgh = [[ecg[2 * h:2 * h + 2, s].reshape(HALF, 1, LAT) for s in range(3)]
            for h in range(2)]
    ncgh = [[ncg[2 * h:2 * h + 2, s].reshape(HALF, 1, LAT) for s in range(3)]
            for h in range(2)]

    e = [None, None]
    u = [None, None]
    eps = [None, None]
    for s in range(3):
        ghr = [None, None]
        aggp = [None, None]
        for h in range(2):
            ghr[h] = _sc_gather(hr[h].reshape(N2, LAT), rcv[h]).reshape(HALF, E, LAT)
        for h in range(2):
            if s == 0:
                u[h], e[h] = _kedge0(d3h[h], ghr[h], hs[h], ecgh[h][s],
                                     eew, We[s], W2e[s], b2e[s])
            else:
                u[h], e[h] = _kedge(e[h], ghr[h], hs[h], ecgh[h][s],
                                    We[s], W2e[s], b2e[s])
        for h in range(2):
            aggp[h] = _sc_scatter(u[h].reshape(E2, LAT), rcv[h]).reshape(2, HALF, N, LAT)
        for h in range(2):
            if s < 2:
                hh[h], hs[h], hr[h] = _knode(hh[h], aggp[h], ncgh[h][s],
                                             A1[s], A2[s], W2n[s], b2n[s],
                                             Ws[s + 1], Wr[s + 1])
            else:
                eps[h] = _knode_final(zh[h], hh[h], aggp[h], ncgh[h][s],
                                      A1[s], A2[s], W2n[s], b2n[s],
                                      d1, db1, d2, db2)
    return jnp.concatenate(eps, axis=0)
